# baseline (speedup 1.0000x reference)
import jax
import jax.numpy as jnp
from jax.experimental import pallas as pl
from jax.experimental.pallas import tpu as pltpu

_NUM_EXPERTS = 64
_EMBED_DIM = 4096
_TOP_K = 2
_BLOCK_ROWS = 1024


def _router_block(x_ref, c_ref, w_ref, i_ref, p_ref):
    x = x_ref[...]
    c = c_ref[...]
    xc = jax.lax.dot_general(
        x, c, (((1,), (1,)), ((), ())),
        preferred_element_type=jnp.float32)
    x_sq = jnp.sum(x * x, axis=-1, keepdims=True)
    c_sq = jnp.sum(c * c, axis=-1)[None, :]
    d2 = jnp.maximum(x_sq + c_sq - 2.0 * xc, 1e-12)
    scaled = -jnp.sqrt(d2)

    m1 = jnp.max(scaled, axis=-1, keepdims=True)
    e = jnp.exp(scaled - m1)
    s = jnp.sum(e, axis=-1, keepdims=True)
    rs = 1.0 / s
    p_ref[...] = e * rs

    col = jax.lax.broadcasted_iota(jnp.int32, scaled.shape, 1)
    i1 = jnp.argmax(scaled, axis=-1, keepdims=True).astype(jnp.int32)
    masked = jnp.where(col == i1, -jnp.inf, scaled)
    m2 = jnp.max(masked, axis=-1, keepdims=True)
    i2 = jnp.argmax(masked, axis=-1, keepdims=True).astype(jnp.int32)
    w1 = rs
    w2 = jnp.exp(m2 - m1) * rs
    denom = jnp.clip(w1 + w2, 1e-6, None)
    w_ref[...] = jnp.concatenate([w1, w2], axis=-1) / denom
    i_ref[...] = jnp.concatenate([i1, i2], axis=-1)


def kernel(embeds, centroids):
    n = embeds.shape[0]
    grid = (n // _BLOCK_ROWS,)
    w, i, p = pl.pallas_call(
        _router_block,
        grid=grid,
        in_specs=[
            pl.BlockSpec((_BLOCK_ROWS, _EMBED_DIM), lambda b: (b, 0)),
            pl.BlockSpec((_NUM_EXPERTS, _EMBED_DIM), lambda b: (0, 0)),
        ],
        out_specs=[
            pl.BlockSpec((_BLOCK_ROWS, _TOP_K), lambda b: (b, 0)),
            pl.BlockSpec((_BLOCK_ROWS, _TOP_K), lambda b: (b, 0)),
            pl.BlockSpec((_BLOCK_ROWS, _NUM_EXPERTS), lambda b: (b, 0)),
        ],
        out_shape=[
            jax.ShapeDtypeStruct((n, _TOP_K), jnp.float32),
            jax.ShapeDtypeStruct((n, _TOP_K), jnp.int32),
            jax.ShapeDtypeStruct((n, _NUM_EXPERTS), jnp.float32),
        ],
        compiler_params=pltpu.CompilerParams(
            dimension_semantics=("parallel",)),
    )(embeds, centroids)
    return (w, i, p)


# speedup vs baseline: 1.0019x; 1.0019x over previous
import jax
import jax.numpy as jnp
from jax.experimental import pallas as pl
from jax.experimental.pallas import tpu as pltpu

_NUM_EXPERTS = 64
_EMBED_DIM = 4096
_TOP_K = 2
_BLOCK_ROWS = 1024


def _router_block(x_ref, c_ref, w_ref, i_ref, p_ref, csq_ref):
    @pl.when(pl.program_id(0) == 0)
    def _init():
        c0 = c_ref[...]
        csq_ref[...] = jnp.sum(c0 * c0, axis=-1)[None, :]

    x = x_ref[...]
    xc = jax.lax.dot_general(
        x, c_ref[...], (((1,), (1,)), ((), ())),
        preferred_element_type=jnp.float32)
    x_sq = jnp.sum(x * x, axis=-1, keepdims=True)
    c_sq = csq_ref[...]
    d2 = jnp.maximum(x_sq + c_sq - 2.0 * xc, 1e-12)
    scaled = -jnp.sqrt(d2)

    m1 = jnp.max(scaled, axis=-1, keepdims=True)
    e = jnp.exp(scaled - m1)
    s = jnp.sum(e, axis=-1, keepdims=True)
    rs = 1.0 / s
    p_ref[...] = e * rs

    col = jax.lax.broadcasted_iota(jnp.int32, scaled.shape, 1)
    i1 = jnp.argmax(scaled, axis=-1, keepdims=True).astype(jnp.int32)
    masked = jnp.where(col == i1, -jnp.inf, scaled)
    m2 = jnp.max(masked, axis=-1, keepdims=True)
    i2 = jnp.argmax(masked, axis=-1, keepdims=True).astype(jnp.int32)
    w1 = rs
    w2 = jnp.exp(m2 - m1) * rs
    denom = jnp.clip(w1 + w2, 1e-6, None)
    w_ref[...] = jnp.concatenate([w1, w2], axis=-1) / denom
    i_ref[...] = jnp.concatenate([i1, i2], axis=-1)


def kernel(embeds, centroids):
    n = embeds.shape[0]
    grid = (n // _BLOCK_ROWS,)
    w, i, p = pl.pallas_call(
        _router_block,
        grid=grid,
        in_specs=[
            pl.BlockSpec((_BLOCK_ROWS, _EMBED_DIM), lambda b: (b, 0)),
            pl.BlockSpec((_NUM_EXPERTS, _EMBED_DIM), lambda b: (0, 0)),
        ],
        out_specs=[
            pl.BlockSpec((_BLOCK_ROWS, _TOP_K), lambda b: (b, 0)),
            pl.BlockSpec((_BLOCK_ROWS, _TOP_K), lambda b: (b, 0)),
            pl.BlockSpec((_BLOCK_ROWS, _NUM_EXPERTS), lambda b: (b, 0)),
        ],
        out_shape=[
            jax.ShapeDtypeStruct((n, _TOP_K), jnp.float32),
            jax.ShapeDtypeStruct((n, _TOP_K), jnp.int32),
            jax.ShapeDtypeStruct((n, _NUM_EXPERTS), jnp.float32),
        ],
        scratch_shapes=[
            pltpu.VMEM((1, _NUM_EXPERTS), jnp.float32),
        ],
    )(embeds, centroids)
    return (w, i, p)
